# baseline (speedup 1.0000x reference)
import functools

import jax
import jax.numpy as jnp
from jax import lax
from jax.experimental import pallas as pl
from jax.experimental.pallas import tpu as pltpu
from jax.experimental.pallas import tpu_sc as plsc

B = 16384
L = 200
D = 64
NUM_CORES = 2
NUM_SUBCORES = 16
NUM_WORKERS = NUM_CORES * NUM_SUBCORES
BW = B // NUM_WORKERS
HALF = BW // 2
LPS = 20
NUM_SUPERS = L // LPS

_mesh = plsc.VectorSubcoreMesh(core_axis_name="c", subcore_axis_name="s")


@functools.partial(
    pl.kernel,
    mesh=_mesh,
    out_type=jax.ShapeDtypeStruct((L, D, B), jnp.float32),
    scratch_types=[
        pltpu.VMEM((LPS, BW), jnp.int32),
        pltpu.VMEM((LPS, BW), jnp.int32),
        pltpu.VMEM((BW, D), jnp.float32),
        pltpu.VMEM((BW, D), jnp.float32),
        pltpu.VMEM((D, HALF), jnp.float32),
        pltpu.VMEM((D, HALF), jnp.float32),
        pltpu.SemaphoreType.DMA,
        pltpu.SemaphoreType.DMA,
        pltpu.SemaphoreType.DMA,
        pltpu.SemaphoreType.DMA,
    ],
    compiler_params=pltpu.CompilerParams(use_tc_tiling_on_sc=False,
                                         needs_layout_passes=False),
)
def _emb_lookup_t(idx_hbm, table_hbm, out_hbm,
                  idx_v0, idx_v1, gath0, gath1, tr0, tr1, sg0, sg1, ss0, ss1):
    wid = lax.axis_index("s") * NUM_CORES + lax.axis_index("c")
    wb = wid * BW
    idxb = (idx_v0, idx_v1)
    gath = (gath0, gath1)
    trans = (tr0, tr1)
    sg = (sg0, sg1)
    ss = (ss0, ss1)
    lane = lax.iota(jnp.int32, 16)

    def gather_start(sb, li_local, g):
        pltpu.async_copy(table_hbm.at[idxb[sb].at[li_local]], gath[g], sg[g])

    def gather_wait(g):
        pltpu.make_async_copy(
            table_hbm.at[idx_v0.at[0]], gath[g], sg[g]).wait()

    def store_start(l, h):
        pltpu.async_copy(
            trans[h], out_hbm.at[l, :, pl.ds(wb + h * HALF, HALF)], ss[h])

    def store_wait(h):
        pltpu.make_async_copy(
            trans[h], out_hbm.at[0, :, pl.ds(wb, HALF)], ss[h]).wait()

    def transpose_half(g, h):
        def body(i, _):
            b16 = i // (D // 8)
            d0 = (i % (D // 8)) * 8
            row = h * HALF + b16 * 16 + lane
            col0 = b16 * 16
            for dd in range(8):
                d = d0 + dd
                v = plsc.load_gather(
                    gath[g], [row, jnp.full((16,), 0, jnp.int32) + d])
                trans[h][d, pl.ds(col0, 16)] = v
            return 0

        lax.fori_loop(0, (HALF // 16) * (D // 8), body, 0)

    def do_l(l, g):
        for h in (0, 1):
            store_wait(h)
            transpose_half(g, h)
            store_start(l, h)

    store_start(0, 0)
    store_start(0, 1)

    def super_pair(sp, _):
        for sb in (0, 1):
            s = 2 * sp + sb
            l0 = s * LPS
            pltpu.sync_copy(
                idx_hbm.at[pl.ds(l0, LPS), pl.ds(wb, BW)], idxb[sb])
            gather_start(sb, 0, 0)

            def li_pair(p, _):
                for g in (0, 1):
                    li = 2 * p + g
                    gather_wait(g)
                    gather_start(sb, li + 1, 1 - g)
                    do_l(l0 + li, g)
                return 0

            lax.fori_loop(0, (LPS - 2) // 2, li_pair, 0)
            gather_wait(0)
            gather_start(sb, LPS - 1, 1)
            do_l(l0 + LPS - 2, 0)
            gather_wait(1)
            do_l(l0 + LPS - 1, 1)
        return 0

    lax.fori_loop(0, NUM_SUPERS // 2, super_pair, 0)
    store_wait(0)
    store_wait(1)


def kernel(indices, table):
    out_t = _emb_lookup_t(indices.T, table)
    return out_t.transpose(2, 0, 1)


# speedup vs baseline: 2.2519x; 2.2519x over previous
import functools

import jax
import jax.numpy as jnp
from jax import lax
from jax.experimental import pallas as pl
from jax.experimental.pallas import tpu as pltpu
from jax.experimental.pallas import tpu_sc as plsc

B = 16384
L = 200
D = 64
NUM_CORES = 2
NUM_SUBCORES = 16
NUM_WORKERS = NUM_CORES * NUM_SUBCORES
SEQ_PER_W = B // NUM_WORKERS
SEQ_CB = 4
CHUNKS_PER_W = SEQ_PER_W // SEQ_CB
SPS = 32
CPS = SPS // SEQ_CB
NUM_SUPERS = SEQ_PER_W // SPS

_mesh = plsc.VectorSubcoreMesh(core_axis_name="c", subcore_axis_name="s")


@functools.partial(
    pl.kernel,
    mesh=_mesh,
    out_type=jax.ShapeDtypeStruct((B, L, D), jnp.float32),
    scratch_types=[
        pltpu.VMEM((SPS, L), jnp.int32),
        pltpu.VMEM((SPS, L), jnp.int32),
        pltpu.VMEM((SEQ_CB, L, D), jnp.float32),
        pltpu.VMEM((SEQ_CB, L, D), jnp.float32),
        pltpu.SemaphoreType.DMA,
        pltpu.SemaphoreType.DMA,
        pltpu.SemaphoreType.DMA,
        pltpu.SemaphoreType.DMA,
    ],
    compiler_params=pltpu.CompilerParams(use_tc_tiling_on_sc=False),
)
def _emb_lookup(idx_hbm, table_hbm, out_hbm,
                idx_v0, idx_v1, rows0, rows1, sg0, sg1, ss0, ss1):
    wid = lax.axis_index("s") * NUM_CORES + lax.axis_index("c")
    wseq = wid * SEQ_PER_W
    idxb = (idx_v0, idx_v1)
    rows = (rows0, rows1)
    sg = (sg0, sg1)
    ss = (ss0, ss1)

    def gather_start(sb, local_chunk, b):
        for jj in range(SEQ_CB):
            pltpu.async_copy(
                table_hbm.at[idxb[sb].at[local_chunk * SEQ_CB + jj]],
                rows[b].at[jj], sg[b])

    def gather_wait(b):
        for jj in range(SEQ_CB):
            pltpu.make_async_copy(
                table_hbm.at[idx_v0.at[0]], rows[b].at[jj], sg[b]).wait()

    def store_start(seq0, b):
        pltpu.async_copy(rows[b], out_hbm.at[pl.ds(seq0, SEQ_CB)], ss[b])

    def store_wait(b):
        pltpu.make_async_copy(
            rows[b], out_hbm.at[pl.ds(wseq, SEQ_CB)], ss[b]).wait()

    pltpu.sync_copy(idx_hbm.at[pl.ds(wseq, SPS)], idx_v0)
    gather_start(0, 0, 1)
    pltpu.async_copy(rows0, out_hbm.at[pl.ds(wseq, SEQ_CB)], ss0)

    def super_pair(sp, _):
        for sb in (0, 1):
            s = 2 * sp + sb
            pltpu.sync_copy(idx_hbm.at[pl.ds(wseq + s * SPS, SPS)], idxb[sb])

            def chunk_pair(p, _):
                for b in (0, 1):
                    lc = 2 * p + b
                    c = s * CPS + lc
                    store_wait(b)
                    gather_start(sb, lc, b)
                    gather_wait(1 - b)
                    prev = wseq + jnp.maximum(c - 1, 0) * SEQ_CB
                    store_start(prev, 1 - b)
                return 0

            lax.fori_loop(0, CPS // 2, chunk_pair, 0)
        return 0

    lax.fori_loop(0, NUM_SUPERS // 2, super_pair, 0)

    gather_wait(1)
    store_start(wseq + (CHUNKS_PER_W - 1) * SEQ_CB, 1)
    store_wait(0)
    store_wait(1)


def kernel(indices, table):
    return _emb_lookup(indices, table)


# speedup vs baseline: 2.3976x; 1.0647x over previous
import functools

import jax
import jax.numpy as jnp
from jax import lax
from jax.experimental import pallas as pl
from jax.experimental.pallas import tpu as pltpu
from jax.experimental.pallas import tpu_sc as plsc

B = 16384
L = 200
D = 64
NUM_CORES = 2
NUM_SUBCORES = 16
NUM_WORKERS = NUM_CORES * NUM_SUBCORES
BW = B // NUM_WORKERS
LPS = 20
NUM_SUPERS = L // LPS

_mesh = plsc.VectorSubcoreMesh(core_axis_name="c", subcore_axis_name="s")


@functools.partial(
    pl.kernel,
    mesh=_mesh,
    out_type=jax.ShapeDtypeStruct((L, B, D), jnp.float32),
    scratch_types=[
        pltpu.VMEM((LPS, BW), jnp.int32),
        pltpu.VMEM((LPS, BW), jnp.int32),
        pltpu.VMEM((BW, D), jnp.float32),
        pltpu.VMEM((BW, D), jnp.float32),
        pltpu.SemaphoreType.DMA,
        pltpu.SemaphoreType.DMA,
        pltpu.SemaphoreType.DMA,
        pltpu.SemaphoreType.DMA,
    ],
    compiler_params=pltpu.CompilerParams(use_tc_tiling_on_sc=False),
)
def _emb_lookup_lbd(idx_hbm, table_hbm, out_hbm,
                    idx_v0, idx_v1, rows0, rows1, sg0, sg1, ss0, ss1):
    wid = lax.axis_index("s") * NUM_CORES + lax.axis_index("c")
    wb = wid * BW
    idxb = (idx_v0, idx_v1)
    rows = (rows0, rows1)
    sg = (sg0, sg1)
    ss = (ss0, ss1)

    def gather_start(sb, li_local, b):
        pltpu.async_copy(
            table_hbm.at[idxb[sb].at[li_local]], rows[b], sg[b])

    def gather_wait(b):
        pltpu.make_async_copy(
            table_hbm.at[idx_v0.at[0]], rows[b], sg[b]).wait()

    def store_start(l, b):
        pltpu.async_copy(rows[b], out_hbm.at[l, pl.ds(wb, BW)], ss[b])

    def store_wait(b):
        pltpu.make_async_copy(
            rows[b], out_hbm.at[0, pl.ds(wb, BW)], ss[b]).wait()

    pltpu.sync_copy(idx_hbm.at[pl.ds(0, LPS), pl.ds(wb, BW)], idx_v0)
    gather_start(0, 0, 1)
    store_start(0, 0)

    def super_pair(sp, _):
        for sb in (0, 1):
            s = 2 * sp + sb
            pltpu.sync_copy(
                idx_hbm.at[pl.ds(s * LPS, LPS), pl.ds(wb, BW)], idxb[sb])

            def pos_pair(p, _):
                for b in (0, 1):
                    li = 2 * p + b
                    l = s * LPS + li
                    store_wait(b)
                    gather_start(sb, li, b)
                    gather_wait(1 - b)
                    store_start(jnp.maximum(l - 1, 0), 1 - b)
                return 0

            lax.fori_loop(0, LPS // 2, pos_pair, 0)
        return 0

    lax.fori_loop(0, NUM_SUPERS // 2, super_pair, 0)

    gather_wait(1)
    store_start(L - 1, 1)
    store_wait(0)
    store_wait(1)


def kernel(indices, table):
    out_lbd = _emb_lookup_lbd(indices.T, table)
    return out_lbd.transpose(1, 0, 2)


# speedup vs baseline: 2.7758x; 1.1578x over previous
import functools

import jax
import jax.numpy as jnp
from jax import lax
from jax.experimental import pallas as pl
from jax.experimental.pallas import tpu as pltpu
from jax.experimental.pallas import tpu_sc as plsc

B = 16384
L = 200
D = 64
NUM_CORES = 2
NUM_SUBCORES = 16
NUM_WORKERS = NUM_CORES * NUM_SUBCORES
BW = B // NUM_WORKERS
FW = BW // 2
LPS = 20
NUM_SUPERS = L // LPS

_mesh = plsc.VectorSubcoreMesh(core_axis_name="c", subcore_axis_name="s")


@functools.partial(
    pl.kernel,
    mesh=_mesh,
    out_type=jax.ShapeDtypeStruct((L, B // 2, 2 * D), jnp.float32),
    scratch_types=[
        pltpu.VMEM((LPS, 2, FW), jnp.int32),
        pltpu.VMEM((LPS, 2, FW), jnp.int32),
        pltpu.VMEM((2, FW, D), jnp.float32),
        pltpu.VMEM((2, FW, D), jnp.float32),
        pltpu.SemaphoreType.DMA,
        pltpu.SemaphoreType.DMA,
        pltpu.SemaphoreType.DMA,
        pltpu.SemaphoreType.DMA,
    ],
    compiler_params=pltpu.CompilerParams(use_tc_tiling_on_sc=False),
)
def _emb_lookup_fused(idx_hbm, table_hbm, out_hbm,
                      idx_v0, idx_v1, rows0, rows1, sg0, sg1, ss0, ss1):
    wid = lax.axis_index("s") * NUM_CORES + lax.axis_index("c")
    wf = wid * FW
    idxb = (idx_v0, idx_v1)
    rows = (rows0, rows1)
    sg = (sg0, sg1)
    ss = (ss0, ss1)

    def gather_start(sb, li_local, b):
        for eo in range(2):
            pltpu.async_copy(
                table_hbm.at[idxb[sb].at[li_local, eo]],
                rows[b].at[eo], sg[b])

    def gather_wait(b):
        for eo in range(2):
            pltpu.make_async_copy(
                table_hbm.at[idx_v0.at[0, 0]], rows[b].at[eo], sg[b]).wait()

    def store_start(l, b):
        for eo in range(2):
            pltpu.async_copy(
                rows[b].at[eo],
                out_hbm.at[l, pl.ds(wf, FW), pl.ds(eo * D, D)], ss[b])

    def store_wait(b):
        for eo in range(2):
            pltpu.make_async_copy(
                rows[b].at[eo],
                out_hbm.at[0, pl.ds(wf, FW), pl.ds(eo * D, D)], ss[b]).wait()

    pltpu.sync_copy(
        idx_hbm.at[pl.ds(0, LPS), :, pl.ds(wf, FW)], idx_v0)
    gather_start(0, 0, 1)
    store_start(0, 0)

    def super_pair(sp, _):
        for sb in (0, 1):
            s = 2 * sp + sb
            pltpu.sync_copy(
                idx_hbm.at[pl.ds(s * LPS, LPS), :, pl.ds(wf, FW)], idxb[sb])

            def pos_pair(p, _):
                for b in (0, 1):
                    li = 2 * p + b
                    l = s * LPS + li
                    store_wait(b)
                    gather_start(sb, li, b)
                    gather_wait(1 - b)
                    store_start(jnp.maximum(l - 1, 0), 1 - b)
                return 0

            lax.fori_loop(0, LPS // 2, pos_pair, 0)
        return 0

    lax.fori_loop(0, NUM_SUPERS // 2, super_pair, 0)

    gather_wait(1)
    store_start(L - 1, 1)
    store_wait(0)
    store_wait(1)


def kernel(indices, table):
    idx_eo = indices.T.reshape(L, B // 2, 2).transpose(0, 2, 1)
    out_f = _emb_lookup_fused(idx_eo, table)
    return out_f.reshape(L, B, D).transpose(1, 0, 2)
